# baseline (speedup 1.0000x reference)
import functools

import jax
import jax.numpy as jnp
from jax import lax
from jax.experimental import pallas as pl
from jax.experimental.pallas import tpu as pltpu
from jax.experimental.pallas import tpu_sc as plsc

V_SIZE = 1000000
EMB_SIZE = 64
BATCH = 16384

NC = 2
NS = 16
LANES = 16
NW = NC * NS
B_PER_W = BATCH // NW
CHUNK = 128
N_CHUNKS = B_PER_W // CHUNK


def _sc_partials(focus, context, emb):
  mesh = plsc.VectorSubcoreMesh(core_axis_name="c", subcore_axis_name="s")

  @functools.partial(
      pl.kernel,
      out_type=jax.ShapeDtypeStruct((NW, LANES), jnp.float32),
      mesh=mesh,
      compiler_params=pltpu.CompilerParams(use_tc_tiling_on_sc=False),
      scratch_types=[
          pltpu.VMEM((B_PER_W,), jnp.int32),
          pltpu.VMEM((B_PER_W,), jnp.int32),
          pltpu.VMEM((B_PER_W, EMB_SIZE), jnp.float32),
          pltpu.VMEM((B_PER_W, EMB_SIZE), jnp.float32),
          pltpu.VMEM((LANES,), jnp.float32),
          pltpu.SemaphoreType.DMA,
      ],
  )
  def body(focus_hbm, ctx_hbm, emb_hbm, out_hbm,
           idx_f, idx_c, rows_f, rows_c, partial_v, sem):
    wid = lax.axis_index("s") * NC + lax.axis_index("c")
    base = wid * B_PER_W

    pltpu.sync_copy(focus_hbm.at[pl.ds(base, B_PER_W)], idx_f)
    pltpu.sync_copy(ctx_hbm.at[pl.ds(base, B_PER_W)], idx_c)

    copies = []
    for c in range(N_CHUNKS):
      sl = pl.ds(c * CHUNK, CHUNK)
      copies.append(
          pltpu.async_copy(emb_hbm.at[idx_f.at[sl]], rows_f.at[sl], sem))
      copies.append(
          pltpu.async_copy(emb_hbm.at[idx_c.at[sl]], rows_c.at[sl], sem))
    for cp in copies:
      cp.wait()

    n_sub = EMB_SIZE // LANES

    def row_body(i, accs):
      return tuple(
          accs[j] + rows_f[i, pl.ds(j * LANES, LANES)]
          * rows_c[i, pl.ds(j * LANES, LANES)]
          for j in range(n_sub))

    zero = jnp.zeros((LANES,), jnp.float32)
    accs = lax.fori_loop(0, B_PER_W, row_body, (zero,) * n_sub)
    total = accs[0]
    for j in range(1, n_sub):
      total = total + accs[j]
    partial_v[...] = total
    pltpu.sync_copy(partial_v, out_hbm.at[wid])

  return body(focus, context, emb)


def _finalize(partials):
  def tc_body(p_ref, o_ref):
    s = jnp.sum(p_ref[...])
    ls = jnp.minimum(s, 0.0) - jnp.log(1.0 + jnp.exp(-jnp.abs(s)))
    o_ref[...] = jnp.reshape(ls, (1, 1))

  return pl.pallas_call(
      tc_body,
      out_shape=jax.ShapeDtypeStruct((1, 1), jnp.float32),
  )(partials)


@jax.jit
def kernel(focus, context, emb):
  partials = _sc_partials(focus, context, emb)
  return _finalize(partials)
